# Initial kernel scaffold; baseline (speedup 1.0000x reference)
import jax
import jax.numpy as jnp
from jax.experimental import pallas as pl


def kernel(input, embedding):
    raise NotImplementedError("write your pallas kernel here")


# speedup vs baseline: 1.1887x; 1.1887x over previous
import functools

import jax
import jax.numpy as jnp
from jax import lax
from jax.experimental import pallas as pl
from jax.experimental.pallas import tpu as pltpu
from jax.experimental.pallas import tpu_sc as plsc

NUM_CODE = 8192
CODE_DIM = 256
M_BLK = 2048
K_BLK = 1024


def _argmin_body(f_ref, et_ref, idx_ref, bestd_ref, besti_ref):
    c = pl.program_id(1)
    nc = pl.num_programs(1)
    f = f_ref[...]
    et = et_ref[...]
    s = jnp.dot(f, et, preferred_element_type=jnp.float32)
    d = 2.0 - 2.0 * s
    lmin = jnp.min(d, axis=1, keepdims=True)
    ids = lax.broadcasted_iota(jnp.int32, d.shape, 1)
    larg = jnp.min(jnp.where(d == lmin, ids, jnp.int32(2**30)),
                   axis=1, keepdims=True) + c * K_BLK

    @pl.when(c == 0)
    def _():
        bestd_ref[...] = lmin
        besti_ref[...] = larg

    @pl.when(c > 0)
    def _():
        better = lmin < bestd_ref[...]
        bestd_ref[...] = jnp.where(better, lmin, bestd_ref[...])
        besti_ref[...] = jnp.where(better, larg, besti_ref[...])

    @pl.when(c == nc - 1)
    def _():
        idx_ref[...] = besti_ref[...]


def _argmin_indices(fb, ebT):
    R, N = fb.shape
    grid = (R // M_BLK, NUM_CODE // K_BLK)
    return pl.pallas_call(
        _argmin_body,
        grid=grid,
        in_specs=[
            pl.BlockSpec((M_BLK, N), lambda m, c: (m, 0)),
            pl.BlockSpec((N, K_BLK), lambda m, c: (0, c)),
        ],
        out_specs=pl.BlockSpec((M_BLK, 1), lambda m, c: (m, 0)),
        out_shape=jax.ShapeDtypeStruct((R, 1), jnp.int32),
        scratch_shapes=[pltpu.VMEM((M_BLK, 1), jnp.float32),
                        pltpu.VMEM((M_BLK, 1), jnp.int32)],
    )(fb, ebT)


def _sc_gather(table, idx_flat):
    info = plsc.get_sparse_core_info()
    NC, NS = info.num_cores, info.num_subcores
    NW = NC * NS
    R = idx_flat.shape[0]
    per_w = R // NW
    CH = 128
    n_ch = per_w // CH
    D = table.shape[1]
    mesh = plsc.VectorSubcoreMesh(core_axis_name="c", subcore_axis_name="s")

    @functools.partial(
        pl.kernel,
        mesh=mesh,
        out_type=jax.ShapeDtypeStruct((R, D), jnp.float32),
        scratch_types=[
            pltpu.VMEM((CH,), jnp.int32),
            pltpu.VMEM((CH, D), jnp.float32),
            pltpu.SemaphoreType.DMA,
        ],
    )
    def k(table_hbm, idx_hbm, out_hbm, idx_v, rows_v, sem):
        wid = lax.axis_index("s") * NC + lax.axis_index("c")
        base = wid * per_w
        for j in range(n_ch):
            off = base + j * CH
            pltpu.sync_copy(idx_hbm.at[pl.ds(off, CH)], idx_v)
            pltpu.async_copy(table_hbm.at[idx_v], rows_v, sem).wait()
            pltpu.sync_copy(rows_v, out_hbm.at[pl.ds(off, CH)])

    return k(table, idx_flat)


def _transpose_body(g_ref, out_ref):
    out_ref[0] = g_ref[0].T


def _bt_transpose(g, B, N, T):
    return pl.pallas_call(
        _transpose_body,
        grid=(B,),
        in_specs=[pl.BlockSpec((1, T, N), lambda b: (b, 0, 0))],
        out_specs=pl.BlockSpec((1, N, T), lambda b: (b, 0, 0)),
        out_shape=jax.ShapeDtypeStruct((B, N, T), jnp.float32),
    )(g)


def kernel(input, embedding):
    B, N, T = input.shape
    fb = jnp.transpose(input, (0, 2, 1)).reshape(B * T, N).astype(jnp.bfloat16)
    ebT = jnp.transpose(embedding.astype(jnp.bfloat16))
    idx = _argmin_indices(fb, ebT)
    idx_flat = idx.reshape(B * T)
    g = _sc_gather(embedding, idx_flat)
    return _bt_transpose(g.reshape(B, T, N), B, N, T)
